# baseline (speedup 1.0000x reference)
import functools

import jax
import jax.numpy as jnp
from jax import lax
from jax.experimental import pallas as pl
from jax.experimental.pallas import tpu as pltpu
from jax.experimental.pallas import tpu_sc as plsc

NU = 50000
NI = 50000
D = 32
DH = 16
E = 1_600_000

NTILES = 16
HALF = 50176
NFULL = 2 * HALF
PAD_DST = 50000
CH = 128
K = 8
ROWS = 12544
E_PAD = ROWS * CH
RPT = ROWS // NTILES
NBLK = RPT // K
KCH = K * CH
NBUF = 4
RPT_N = HALF // NTILES
ZROWS = 392

_mesh = plsc.VectorSubcoreMesh(core_axis_name="c", subcore_axis_name="s")
_sc_params = pltpu.CompilerParams(use_tc_tiling_on_sc=False)


@functools.partial(
    pl.kernel,
    out_type=(jax.ShapeDtypeStruct((HALF,), jnp.float32),
              jax.ShapeDtypeStruct((HALF,), jnp.float32)),
    mesh=_mesh,
    scratch_types=[
        pltpu.VMEM((4, KCH), jnp.int32),
        pltpu.VMEM((KCH,), jnp.float32),
        pltpu.VMEM((RPT_N,), jnp.float32),
        pltpu.VMEM_SHARED((HALF,), jnp.float32),
        pltpu.SemaphoreType.DMA,
    ],
    compiler_params=_sc_params,
)
def _deg_kernel(gdst, deg_out0, deg_out1, didx, ones_v, zs, accum, ssem):
    c = lax.axis_index("c")
    s = lax.axis_index("s")

    def orow(i, carry):
        ones_v[pl.ds(i * 16, 16)] = jnp.ones((16,), jnp.float32)
        return carry

    lax.fori_loop(0, KCH // 16, orow, 0)

    def zrow(i, carry):
        zs[pl.ds(i * 16, 16)] = jnp.zeros((16,), jnp.float32)
        return carry

    lax.fori_loop(0, RPT_N // 16, zrow, 0)
    pltpu.sync_copy(zs, accum.at[pl.ds(s * RPT_N, RPT_N)])
    plsc.subcore_barrier()

    ebase = s * RPT * CH
    for p in range(3):
        pltpu.sync_copy(gdst.at[c, pl.ds(ebase + p * KCH, KCH)], didx.at[p])

    def block(b, carry):
        cur = lax.rem(b, 4)
        prv = lax.rem(b + 3, 4)
        pltpu.async_copy(ones_v, accum.at[didx.at[cur]], ssem, add=True)

        @pl.when(b >= 1)
        def _():
            pltpu.make_async_copy(ones_v, accum.at[didx.at[prv]], ssem).wait()

        @pl.when(b + 3 < NBLK)
        def _():
            pltpu.sync_copy(gdst.at[c, pl.ds(ebase + (b + 3) * KCH, KCH)],
                            didx.at[prv])
        return carry

    lax.fori_loop(0, NBLK, block, 0)
    pltpu.make_async_copy(ones_v, accum.at[didx.at[(NBLK - 1) % 4]],
                          ssem).wait()
    plsc.subcore_barrier()
    pltpu.sync_copy(accum.at[pl.ds(s * RPT_N, RPT_N)], zs)

    @pl.when(c == 0)
    def _():
        pltpu.sync_copy(zs, deg_out0.at[pl.ds(s * RPT_N, RPT_N)])

    @pl.when(c == 1)
    def _():
        pltpu.sync_copy(zs, deg_out1.at[pl.ds(s * RPT_N, RPT_N)])


@functools.partial(
    pl.kernel,
    out_type=jax.ShapeDtypeStruct((2, 2, HALF, DH), jnp.float32),
    mesh=_mesh,
    scratch_types=[
        pltpu.VMEM((NBUF, KCH), jnp.int32),
        pltpu.VMEM((NBUF, KCH), jnp.int32),
        pltpu.VMEM((NBUF, KCH, DH), jnp.float32),
        pltpu.VMEM((ZROWS, DH), jnp.float32),
        pltpu.VMEM_SHARED((HALF, DH), jnp.float32),
        pltpu.SemaphoreType.DMA,
        pltpu.SemaphoreType.DMA,
    ],
    compiler_params=_sc_params,
)
def _prop_kernel(y, gsrc, gdst, s_out, sidx, didx, rows, zs, accum,
                 gsem, ssem):
    c = lax.axis_index("c")
    s = lax.axis_index("s")
    ebase = s * RPT * CH

    def phase(h, carry):
        yh = y.at[h]
        def zrow(i, carry2):
            zs[i, pl.ds(0, 16)] = jnp.zeros((16,), jnp.float32)
            return carry2

        lax.fori_loop(0, ZROWS, zrow, 0)

        def zcopy(k, carry2):
            pltpu.sync_copy(zs,
                            accum.at[pl.ds(s * RPT_N + k * ZROWS, ZROWS)])
            return carry2

        lax.fori_loop(0, RPT_N // ZROWS, zcopy, 0)
        plsc.subcore_barrier()

        def stage(slot, e0):
            pltpu.sync_copy(gsrc.at[c, pl.ds(e0, KCH)], sidx.at[slot])
            pltpu.sync_copy(gdst.at[c, pl.ds(e0, KCH)], didx.at[slot])

        def fire_gather(slot):
            pltpu.async_copy(yh.at[sidx.at[slot]], rows.at[slot], gsem)

        for p in range(NBUF - 1):
            stage(p, ebase + p * KCH)
            fire_gather(p)

        def block(b, carry2):
            cur = lax.rem(b, NBUF)
            prv = lax.rem(b + NBUF - 1, NBUF)
            pltpu.make_async_copy(yh.at[sidx.at[cur]], rows.at[cur],
                                  gsem).wait()
            pltpu.async_copy(rows.at[cur], accum.at[didx.at[cur]], ssem,
                             add=True)

            @pl.when(b >= 1)
            def _():
                pltpu.make_async_copy(rows.at[prv], accum.at[didx.at[prv]],
                                      ssem).wait()

            @pl.when(b + NBUF - 1 < NBLK)
            def _():
                stage(prv, ebase + (b + NBUF - 1) * KCH)
                fire_gather(prv)
            return carry2

        lax.fori_loop(0, NBLK, block, 0)
        last = (NBLK - 1) % NBUF
        pltpu.make_async_copy(rows.at[last], accum.at[didx.at[last]],
                              ssem).wait()
        plsc.subcore_barrier()

        def chunk(k, carry2):
            off = s * RPT_N + k * ZROWS
            pltpu.sync_copy(accum.at[pl.ds(off, ZROWS)], zs)
            pltpu.sync_copy(zs, s_out.at[h, c, pl.ds(off, ZROWS)])
            return carry2

        lax.fori_loop(0, RPT_N // ZROWS, chunk, 0)
        return carry

    lax.fori_loop(0, 2, phase, 0)


_PB = NFULL // 128
_QB = NFULL * DH // 128
_UB = HALF * DH // 128


def _norm_body(deg_ref, n_ref, n2_ref):
    d = deg_ref[...]
    n_ref[...] = jnp.where(d > 0, lax.rsqrt(jnp.maximum(d, 1.0)), 0.0)
    n2_ref[...] = jnp.where(d > 0, 1.0 / jnp.maximum(d, 1.0), 0.0)


_norm_k = pl.pallas_call(
    _norm_body,
    out_shape=[jax.ShapeDtypeStruct((_PB, 128), jnp.float32)] * 2,
)


def _prep_body(x_ref, n_ref, y_ref):
    y_ref[...] = x_ref[...] * n_ref[...]


_prep_k = pl.pallas_call(
    _prep_body,
    grid=(2, 4),
    in_specs=[
        pl.BlockSpec((1, _QB // 4, 128), lambda h, i: (h, i, 0)),
        pl.BlockSpec((_QB // 4, 128), lambda h, i: (i, 0)),
    ],
    out_specs=pl.BlockSpec((1, _QB // 4, 128), lambda h, i: (h, i, 0)),
    out_shape=jax.ShapeDtypeStruct((2, _QB, 128), jnp.float32),
)


def _scale_body(s_ref, xa_ref, n1_ref, n2_ref, y_ref, xn_ref):
    sv = s_ref[...]
    y_ref[...] = sv * n2_ref[...]
    xn_ref[...] = xa_ref[...] + sv * n1_ref[...]


_scale_k = pl.pallas_call(
    _scale_body,
    grid=(2, 4),
    in_specs=[
        pl.BlockSpec((1, _QB // 4, 128), lambda h, i: (h, i, 0)),
        pl.BlockSpec((1, _QB // 4, 128), lambda h, i: (h, i, 0)),
        pl.BlockSpec((_QB // 4, 128), lambda h, i: (i, 0)),
        pl.BlockSpec((_QB // 4, 128), lambda h, i: (i, 0)),
    ],
    out_specs=[pl.BlockSpec((1, _QB // 4, 128), lambda h, i: (h, i, 0))] * 2,
    out_shape=[jax.ShapeDtypeStruct((2, _QB, 128), jnp.float32)] * 2,
)


def _final_body(s_ref, xa_ref, n1_ref, out_ref):
    out_ref[...] = 0.25 * (xa_ref[...] + s_ref[...] * n1_ref[...])


_final_k = pl.pallas_call(
    _final_body,
    grid=(2, 4),
    in_specs=[
        pl.BlockSpec((1, _UB // 4, 128), lambda h, i: (h, i, 0)),
        pl.BlockSpec((1, _UB // 4, 128), lambda h, i: (h, i, 0)),
        pl.BlockSpec((_UB // 4, 128), lambda h, i: (i, 0)),
    ],
    out_specs=pl.BlockSpec((1, _UB // 4, 128), lambda h, i: (h, i, 0)),
    out_shape=jax.ShapeDtypeStruct((2, _UB, 128), jnp.float32),
)


def kernel(edge_index, u_weight, i_weight):
    u = edge_index[0].astype(jnp.int32)
    it = edge_index[1].astype(jnp.int32)
    pad = E_PAD - E
    gsrc = jnp.stack([it + HALF, u])
    gdst = jnp.stack([u, it])
    gsrc = jnp.pad(gsrc, ((0, 0), (0, pad)))
    gdst = jnp.pad(gdst, ((0, 0), (0, pad)), constant_values=PAD_DST)

    deg0, deg1 = _deg_kernel(gdst)
    deg = jnp.concatenate([deg0, deg1]).reshape(_PB, 128)
    norm_p, norm2_p = _norm_k(deg)
    nb1 = jnp.broadcast_to(
        norm_p.reshape(NFULL, 1), (NFULL, DH)).reshape(_QB, 128)
    nb2 = jnp.broadcast_to(
        norm2_p.reshape(NFULL, 1), (NFULL, DH)).reshape(_QB, 128)

    def pack_half(col0):
        return (jnp.zeros((2, HALF, DH), jnp.float32)
                .at[0, :NU].set(u_weight[:, col0:col0 + DH])
                .at[1, :NI].set(i_weight[:, col0:col0 + DH])
                .reshape(_QB, 128))

    xa = jnp.stack([pack_half(0), pack_half(DH)])
    yp = _prep_k(xa, nb1)
    out = None
    for layer in range(3):
        sp = _prop_kernel(yp.reshape(2, NFULL, DH), gsrc, gdst)
        sp = sp.reshape(2, _QB, 128)
        if layer < 2:
            yp, xa = _scale_k(sp, xa, nb1, nb2)
        else:
            o = _final_k(sp, xa, nb1).reshape(2, HALF, DH)
            out = jnp.concatenate([o[0, :NU], o[1, :NU]], axis=1)
    return out


# speedup vs baseline: 1.0162x; 1.0162x over previous
import functools

import jax
import jax.numpy as jnp
from jax import lax
from jax.experimental import pallas as pl
from jax.experimental.pallas import tpu as pltpu
from jax.experimental.pallas import tpu_sc as plsc

NU = 50000
NI = 50000
D = 32
DH = 16
E = 1_600_000

NTILES = 16
HALF = 50176
NFULL = 2 * HALF
PAD_DST = 50000
CH = 128
K = 8
ROWS = 12544
E_PAD = ROWS * CH
RPT = ROWS // NTILES
NBLK = RPT // K
KCH = K * CH
NBUF = 4
RPT_N = HALF // NTILES
ZROWS = 392

_mesh = plsc.VectorSubcoreMesh(core_axis_name="c", subcore_axis_name="s")
_sc_params = pltpu.CompilerParams(use_tc_tiling_on_sc=False)


@functools.partial(
    pl.kernel,
    out_type=(jax.ShapeDtypeStruct((HALF,), jnp.float32),
              jax.ShapeDtypeStruct((HALF,), jnp.float32)),
    mesh=_mesh,
    scratch_types=[
        pltpu.VMEM((4, KCH), jnp.int32),
        pltpu.VMEM((KCH,), jnp.float32),
        pltpu.VMEM((RPT_N,), jnp.float32),
        pltpu.VMEM_SHARED((HALF,), jnp.float32),
        pltpu.SemaphoreType.DMA,
    ],
    compiler_params=_sc_params,
)
def _deg_kernel(gdst, deg_out0, deg_out1, didx, ones_v, zs, accum, ssem):
    c = lax.axis_index("c")
    s = lax.axis_index("s")

    def orow(i, carry):
        ones_v[pl.ds(i * 16, 16)] = jnp.ones((16,), jnp.float32)
        return carry

    lax.fori_loop(0, KCH // 16, orow, 0)

    def zrow(i, carry):
        zs[pl.ds(i * 16, 16)] = jnp.zeros((16,), jnp.float32)
        return carry

    lax.fori_loop(0, RPT_N // 16, zrow, 0)
    pltpu.sync_copy(zs, accum.at[pl.ds(s * RPT_N, RPT_N)])
    plsc.subcore_barrier()

    ebase = s * RPT * CH
    for p in range(3):
        pltpu.sync_copy(gdst.at[c, pl.ds(ebase + p * KCH, KCH)], didx.at[p])

    def block(b, carry):
        cur = lax.rem(b, 4)
        prv = lax.rem(b + 3, 4)
        pltpu.async_copy(ones_v, accum.at[didx.at[cur]], ssem, add=True)

        @pl.when(b >= 1)
        def _():
            pltpu.make_async_copy(ones_v, accum.at[didx.at[prv]], ssem).wait()

        @pl.when(b + 3 < NBLK)
        def _():
            pltpu.sync_copy(gdst.at[c, pl.ds(ebase + (b + 3) * KCH, KCH)],
                            didx.at[prv])
        return carry

    lax.fori_loop(0, NBLK, block, 0)
    pltpu.make_async_copy(ones_v, accum.at[didx.at[(NBLK - 1) % 4]],
                          ssem).wait()
    plsc.subcore_barrier()
    pltpu.sync_copy(accum.at[pl.ds(s * RPT_N, RPT_N)], zs)

    @pl.when(c == 0)
    def _():
        pltpu.sync_copy(zs, deg_out0.at[pl.ds(s * RPT_N, RPT_N)])

    @pl.when(c == 1)
    def _():
        pltpu.sync_copy(zs, deg_out1.at[pl.ds(s * RPT_N, RPT_N)])


@functools.partial(
    pl.kernel,
    out_type=(jax.ShapeDtypeStruct((2, HALF, DH), jnp.float32),
              jax.ShapeDtypeStruct((2, HALF, DH), jnp.float32)),
    mesh=_mesh,
    scratch_types=[
        pltpu.VMEM((NBUF, KCH), jnp.int32),
        pltpu.VMEM((NBUF, KCH), jnp.int32),
        pltpu.VMEM((NBUF, KCH, DH), jnp.float32),
        pltpu.VMEM((ZROWS, DH), jnp.float32),
        pltpu.VMEM_SHARED((HALF, DH), jnp.float32),
        pltpu.SemaphoreType.DMA,
        pltpu.SemaphoreType.DMA,
    ],
    compiler_params=_sc_params,
)
def _prop_kernel(y0, y1, gsrc, gdst, s_out0, s_out1,
                 sidx, didx, rows, zs, accum, gsem, ssem):
    c = lax.axis_index("c")
    s = lax.axis_index("s")
    base = s * RPT

    for yh, sh in ((y0, s_out0), (y1, s_out1)):
        def zrow(i, carry):
            zs[i, pl.ds(0, 16)] = jnp.zeros((16,), jnp.float32)
            return carry

        lax.fori_loop(0, ZROWS, zrow, 0)
        for k in range(RPT_N // ZROWS):
            pltpu.sync_copy(zs, accum.at[pl.ds(s * RPT_N + k * ZROWS, ZROWS)])
        plsc.subcore_barrier()

        ebase = s * RPT * CH

        def stage(slot, e0):
            pltpu.sync_copy(gsrc.at[c, pl.ds(e0, KCH)], sidx.at[slot])
            pltpu.sync_copy(gdst.at[c, pl.ds(e0, KCH)], didx.at[slot])

        def fire_gather(slot, e0):
            pltpu.async_copy(yh.at[sidx.at[slot]], rows.at[slot], gsem)

        for p in range(NBUF - 1):
            stage(p, ebase + p * KCH)
            fire_gather(p, 0)

        def block(b, carry):
            cur = lax.rem(b, NBUF)
            prv = lax.rem(b + NBUF - 1, NBUF)
            pltpu.make_async_copy(yh.at[sidx.at[cur]], rows.at[cur],
                                  gsem).wait()
            pltpu.async_copy(rows.at[cur], accum.at[didx.at[cur]], ssem,
                             add=True)

            @pl.when(b >= 1)
            def _():
                pltpu.make_async_copy(rows.at[prv], accum.at[didx.at[prv]],
                                      ssem).wait()

            @pl.when(b + NBUF - 1 < NBLK)
            def _():
                stage(prv, ebase + (b + NBUF - 1) * KCH)
                fire_gather(prv, 0)
            return carry

        lax.fori_loop(0, NBLK, block, 0)
        last = (NBLK - 1) % NBUF
        pltpu.make_async_copy(rows.at[last], accum.at[didx.at[last]],
                              ssem).wait()
        plsc.subcore_barrier()
        for k in range(RPT_N // ZROWS):
            pltpu.sync_copy(accum.at[pl.ds(s * RPT_N + k * ZROWS, ZROWS)], zs)
            pltpu.sync_copy(zs, sh.at[c, pl.ds(s * RPT_N + k * ZROWS, ZROWS)])


_PB = NFULL // 128
_QB = NFULL * DH // 128
_UB = HALF * DH // 128


def _norm_body(deg_ref, n_ref, n2_ref):
    d = deg_ref[...]
    n_ref[...] = jnp.where(d > 0, lax.rsqrt(jnp.maximum(d, 1.0)), 0.0)
    n2_ref[...] = jnp.where(d > 0, 1.0 / jnp.maximum(d, 1.0), 0.0)


_norm_k = pl.pallas_call(
    _norm_body,
    out_shape=[jax.ShapeDtypeStruct((_PB, 128), jnp.float32)] * 2,
)


def _prep_body(x_ref, n_ref, y_ref):
    y_ref[...] = x_ref[...] * n_ref[...]


_prep_k = pl.pallas_call(
    _prep_body,
    grid=(4,),
    in_specs=[pl.BlockSpec((_QB // 4, 128), lambda i: (i, 0))] * 2,
    out_specs=pl.BlockSpec((_QB // 4, 128), lambda i: (i, 0)),
    out_shape=jax.ShapeDtypeStruct((_QB, 128), jnp.float32),
)


def _scale_body(s_ref, xa_ref, n1_ref, n2_ref, y_ref, xn_ref):
    sv = s_ref[...]
    y_ref[...] = sv * n2_ref[...]
    xn_ref[...] = xa_ref[...] + sv * n1_ref[...]


_scale_k = pl.pallas_call(
    _scale_body,
    grid=(4,),
    in_specs=[pl.BlockSpec((_QB // 4, 128), lambda i: (i, 0))] * 4,
    out_specs=[pl.BlockSpec((_QB // 4, 128), lambda i: (i, 0))] * 2,
    out_shape=[jax.ShapeDtypeStruct((_QB, 128), jnp.float32)] * 2,
)


def _final_body(s_ref, xa_ref, n1_ref, out_ref):
    out_ref[...] = 0.25 * (xa_ref[...] + s_ref[...] * n1_ref[...])


_final_k = pl.pallas_call(
    _final_body,
    grid=(4,),
    in_specs=[pl.BlockSpec((_UB // 4, 128), lambda i: (i, 0))] * 3,
    out_specs=pl.BlockSpec((_UB // 4, 128), lambda i: (i, 0)),
    out_shape=jax.ShapeDtypeStruct((_UB, 128), jnp.float32),
)


def kernel(edge_index, u_weight, i_weight):
    u = edge_index[0].astype(jnp.int32)
    it = edge_index[1].astype(jnp.int32)
    pad = E_PAD - E
    gsrc = jnp.stack([it + HALF, u])
    gdst = jnp.stack([u, it])
    gsrc = jnp.pad(gsrc, ((0, 0), (0, pad)))
    gdst = jnp.pad(gdst, ((0, 0), (0, pad)), constant_values=PAD_DST)

    deg0, deg1 = _deg_kernel(gdst)
    deg = jnp.concatenate([deg0, deg1]).reshape(_PB, 128)
    norm_p, norm2_p = _norm_k(deg)
    nb1 = jnp.broadcast_to(
        norm_p.reshape(NFULL, 1), (NFULL, DH)).reshape(_QB, 128)
    nb2 = jnp.broadcast_to(
        norm2_p.reshape(NFULL, 1), (NFULL, DH)).reshape(_QB, 128)

    def pack_half(col0):
        return (jnp.zeros((2, HALF, DH), jnp.float32)
                .at[0, :NU].set(u_weight[:, col0:col0 + DH])
                .at[1, :NI].set(i_weight[:, col0:col0 + DH])
                .reshape(_QB, 128))

    xa0 = pack_half(0)
    xa1 = pack_half(DH)
    y0p = _prep_k(xa0, nb1)
    y1p = _prep_k(xa1, nb1)
    out = None
    for layer in range(3):
        s0, s1 = _prop_kernel(y0p.reshape(NFULL, DH), y1p.reshape(NFULL, DH),
                              gsrc, gdst)
        s0p = s0.reshape(_QB, 128)
        s1p = s1.reshape(_QB, 128)
        if layer < 2:
            y0p, xa0 = _scale_k(s0p, xa0, nb1, nb2)
            y1p, xa1 = _scale_k(s1p, xa1, nb1, nb2)
        else:
            o0 = _final_k(s0p, xa0, nb1).reshape(HALF, DH)
            o1 = _final_k(s1p, xa1, nb1).reshape(HALF, DH)
            out = jnp.concatenate([o0[:NU], o1[:NU]], axis=1)
    return out
